# baseline (speedup 1.0000x reference)
import functools

import jax
import jax.numpy as jnp
from jax import lax
from jax.experimental import pallas as pl
from jax.experimental.pallas import tpu as pltpu
from jax.experimental.pallas import tpu_sc as plsc

_EPS = 1e-12
_CH = 16


def _rsqrt16(v):
    magic = jnp.full((16,), 0x5F3759DF, jnp.int32)
    vi = lax.bitcast_convert_type(v, jnp.int32)
    seed = magic - lax.shift_right_logical(vi, jnp.full((16,), 1, jnp.int32))
    y = lax.bitcast_convert_type(seed, jnp.float32)
    half = v * 0.5
    for _ in range(3):
        y = y * (1.5 - half * y * y)
    return y


def _sc_body(ids_hbm, tok_hbm, pos_hbm, g_hbm, bt_hbm, out_hbm,
             ids_v, tokrows, posbuf, outbuf, g_v, bt_v, sem,
             *, d, rows_per_w, s_per_w):
    nc = 2
    wid = lax.axis_index("s") * nc + lax.axis_index("c")
    row0 = wid * rows_per_w
    pltpu.sync_copy(ids_hbm.at[pl.ds(row0, rows_per_w)], ids_v)
    pltpu.sync_copy(g_hbm, g_v)
    pltpu.sync_copy(bt_hbm, bt_v)
    nk = d // 16
    nch = rows_per_w // _CH

    def chunk_body(c, carry):
        rbase = row0 + c * _CH
        sbase = rbase % 8192
        cp = pltpu.async_copy(tok_hbm.at[ids_v.at[pl.ds(c * _CH, _CH)]],
                              tokrows, sem)
        pltpu.sync_copy(pos_hbm.at[pl.ds(sbase, _CH)], posbuf)
        cp.wait()

        def row_body(r, carry2):
            acc = jnp.zeros((16,), jnp.float32)
            sq = jnp.zeros((16,), jnp.float32)
            for k in range(nk):
                x = tokrows[r, pl.ds(16 * k, 16)] + posbuf[r, pl.ds(16 * k, 16)]
                outbuf[r, pl.ds(16 * k, 16)] = x
                acc = acc + x
                sq = sq + x * x
            inv_d = 1.0 / d
            iota = lax.iota(jnp.int32, 16)
            for sh in (8, 4, 2, 1):
                rot = lax.rem(iota + sh, jnp.full((16,), 16, jnp.int32))
                acc = acc + acc.at[rot].get(mode="promise_in_bounds")
                sq = sq + sq.at[rot].get(mode="promise_in_bounds")
            mv = acc * inv_d
            ex2 = sq * inv_d
            var = ex2 - mv * mv + _EPS
            rstd = _rsqrt16(var)
            for k in range(nk):
                xc = outbuf[r, pl.ds(16 * k, 16)] - mv
                outbuf[r, pl.ds(16 * k, 16)] = (
                    xc * rstd * g_v[pl.ds(16 * k, 16)] + bt_v[pl.ds(16 * k, 16)])
            return carry2

        lax.fori_loop(0, _CH, row_body, 0)
        pltpu.sync_copy(outbuf, out_hbm.at[pl.ds(rbase, _CH)])
        return carry

    lax.fori_loop(0, nch, chunk_body, 0)


def kernel(input_ids, tok_emb, pos_emb, gamma, beta):
    b, s = input_ids.shape
    vocab, d = tok_emb.shape
    nw = 32
    rows = b * s
    rows_per_w = rows // nw
    s_per_w = rows_per_w

    ids = input_ids.astype(jnp.int32).reshape(rows)
    pos = pos_emb[:s]

    mesh = plsc.VectorSubcoreMesh(core_axis_name="c", subcore_axis_name="s")
    k = functools.partial(
        pl.kernel,
        mesh=mesh,
        out_type=jax.ShapeDtypeStruct((rows, d), jnp.float32),
        scratch_types=[
            pltpu.VMEM((rows_per_w,), jnp.int32),
            pltpu.VMEM((_CH, d), jnp.float32),
            pltpu.VMEM((_CH, d), jnp.float32),
            pltpu.VMEM((_CH, d), jnp.float32),
            pltpu.VMEM((d,), jnp.float32),
            pltpu.VMEM((d,), jnp.float32),
            pltpu.SemaphoreType.DMA,
        ],
    )(functools.partial(_sc_body, d=d, rows_per_w=rows_per_w,
                        s_per_w=s_per_w))
    out = k(ids, tok_emb, pos, gamma, beta)
    return out.reshape(b, s, d)


# speedup vs baseline: 7.8721x; 7.8721x over previous
import functools

import jax
import jax.numpy as jnp
from jax.experimental import pallas as pl

_EPS = 1e-12


def _embed_ln_kernel(ids_ref, tok_ref, pos_ref, gamma_ref, beta_ref, out_ref,
                     *, vocab: int):
    bblk, sblk, _ = ids_ref.shape
    tok_tab = tok_ref[...]
    pos = pos_ref[...]
    g = gamma_ref[...]
    bt = beta_ref[...]
    iota = jax.lax.broadcasted_iota(jnp.int32, (sblk, vocab), 1)
    for bi in range(bblk):
        ids = ids_ref[bi]
        onehot = (ids == iota).astype(jnp.float32)
        x = jnp.dot(onehot, tok_tab, preferred_element_type=jnp.float32) + pos
        mean = jnp.mean(x, axis=-1, keepdims=True)
        xc = x - mean
        var = jnp.mean(xc * xc, axis=-1, keepdims=True)
        xhat = xc * jax.lax.rsqrt(var + _EPS)
        out_ref[bi] = xhat * g + bt


def kernel(input_ids, tok_emb, pos_emb, gamma, beta):
    b, s = input_ids.shape
    vocab, d = tok_emb.shape
    sblk = 1024
    bblk = 2
    grid = (s // sblk, b // bblk)

    ids = input_ids.astype(jnp.int32).reshape(b, s, 1)
    pos = pos_emb[:s]

    out = pl.pallas_call(
        functools.partial(_embed_ln_kernel, vocab=vocab),
        grid=grid,
        in_specs=[
            pl.BlockSpec((bblk, sblk, 1), lambda i, j: (j, i, 0)),
            pl.BlockSpec((vocab, d), lambda i, j: (0, 0)),
            pl.BlockSpec((sblk, d), lambda i, j: (i, 0)),
            pl.BlockSpec((d,), lambda i, j: (0,)),
            pl.BlockSpec((d,), lambda i, j: (0,)),
        ],
        out_specs=pl.BlockSpec((bblk, sblk, d), lambda i, j: (j, i, 0)),
        out_shape=jax.ShapeDtypeStruct((b, s, d), jnp.float32),
    )(ids, tok_emb, pos, gamma, beta)
    return out


# speedup vs baseline: 8.9936x; 1.1425x over previous
import functools

import jax
import jax.numpy as jnp
from jax.experimental import pallas as pl

_EPS = 1e-12


def _embed_ln_kernel(ids_ref, tok_ref, pos_ref, gamma_ref, beta_ref, out_ref,
                     *, vocab: int):
    b, sblk, _ = ids_ref.shape
    d = tok_ref.shape[1]
    tok_tab = tok_ref[...]
    pos = pos_ref[...]
    g = gamma_ref[...]
    bt = beta_ref[...]
    iota = jax.lax.broadcasted_iota(jnp.int32, (sblk, vocab), 1)
    for bi in range(b):
        ids = ids_ref[bi]
        onehot = (ids == iota).astype(jnp.float32)
        x = jnp.dot(onehot, tok_tab, preferred_element_type=jnp.float32) + pos
        mean = jnp.mean(x, axis=-1, keepdims=True)
        ex2 = jnp.mean(x * x, axis=-1, keepdims=True)
        var = ex2 - mean * mean
        xhat = (x - mean) * jax.lax.rsqrt(var + _EPS)
        out_ref[bi] = xhat * g + bt


def kernel(input_ids, tok_emb, pos_emb, gamma, beta):
    b, s = input_ids.shape
    vocab, d = tok_emb.shape
    sblk = 1024
    grid = (s // sblk,)

    ids = input_ids.astype(jnp.int32).reshape(b, s, 1)
    pos = pos_emb[:s]

    out = pl.pallas_call(
        functools.partial(_embed_ln_kernel, vocab=vocab),
        grid=grid,
        in_specs=[
            pl.BlockSpec((b, sblk, 1), lambda i: (0, i, 0)),
            pl.BlockSpec((vocab, d), lambda i: (0, 0)),
            pl.BlockSpec((sblk, d), lambda i: (i, 0)),
            pl.BlockSpec((d,), lambda i: (0,)),
            pl.BlockSpec((d,), lambda i: (0,)),
        ],
        out_specs=pl.BlockSpec((b, sblk, d), lambda i: (0, i, 0)),
        out_shape=jax.ShapeDtypeStruct((b, s, d), jnp.float32),
    )(ids, tok_emb, pos, gamma, beta)
    return out
